# baseline (speedup 1.0000x reference)
import functools

import jax
import jax.numpy as jnp
from jax import lax
from jax.experimental import pallas as pl
from jax.experimental.pallas import tpu as pltpu
from jax.experimental.pallas import tpu_sc as plsc

N_IN = 128
N_HID = 64
N_OUT = 4
D2 = 16
K = 128
NC, NS = 2, 16
NW = NC * NS
BM = 512
QB = 2

_mesh = plsc.VectorSubcoreMesh(
    core_axis_name="c", subcore_axis_name="s", num_cores=NC, num_subcores=NS
)
_sc_params = pltpu.CompilerParams(
    needs_layout_passes=False, use_tc_tiling_on_sc=False
)


def _sc_degree(dst2d, zrow, npad):
    per_w = dst2d.shape[0] // NW

    @functools.partial(
        pl.kernel,
        out_type=jax.ShapeDtypeStruct((NW, npad), jnp.float32),
        mesh=_mesh,
        scratch_types=[
            pltpu.VMEM((per_w, K), jnp.int32),
            pltpu.VMEM((npad,), jnp.float32),
        ],
        compiler_params=_sc_params,
    )
    def deg_kernel(dst_hbm, z_hbm, out_hbm, idx_v, deg_v):
        c = lax.axis_index("c")
        s = lax.axis_index("s")
        w = c * NS + s
        pltpu.sync_copy(dst_hbm.at[pl.ds(w * per_w, per_w)], idx_v)
        pltpu.sync_copy(z_hbm, deg_v)
        ones = jnp.full((16,), 1.0, jnp.float32)

        @pl.loop(0, per_w)
        def _(j):
            for i in range(K // 16):
                plsc.addupdate_scatter(deg_v, [idx_v[j, pl.ds(i * 16, 16)]], ones)

        pltpu.sync_copy(deg_v, out_hbm.at[w])

    return deg_kernel(dst2d, zrow)


def _sc_aggregate(g, src3d, dst3d, zblock):
    nq = src3d.shape[0] // NW
    npad, d = g.shape
    slc = npad // NS

    @functools.partial(
        pl.kernel,
        out_type=jax.ShapeDtypeStruct((NC, npad, d), jnp.float32),
        mesh=_mesh,
        scratch_types=[
            pltpu.VMEM((nq, QB * K), jnp.int32),
            pltpu.VMEM((nq, QB * K), jnp.int32),
            pltpu.VMEM((QB * K, d), jnp.float32),
            pltpu.VMEM_SHARED((npad, d), jnp.float32),
            pltpu.SemaphoreType.DMA,
        ],
        compiler_params=_sc_params,
    )
    def agg_kernel(g_hbm, src_hbm, dst_hbm, z_hbm, out_hbm,
                   sidx_v, didx_v, rows_v, accum, gsem):
        c = lax.axis_index("c")
        s = lax.axis_index("s")
        w = c * NS + s
        pltpu.sync_copy(src_hbm.at[pl.ds(w * nq, nq)], sidx_v)
        pltpu.sync_copy(dst_hbm.at[pl.ds(w * nq, nq)], didx_v)
        pltpu.sync_copy(z_hbm.at[pl.ds(s * slc, slc)],
                        accum.at[pl.ds(s * slc, slc)])
        plsc.subcore_barrier()

        @pl.loop(0, nq)
        def _(j):
            pltpu.async_copy(g_hbm.at[sidx_v.at[j]], rows_v, gsem).wait()
            pltpu.sync_copy(rows_v, accum.at[didx_v.at[j]], add=True)

        plsc.subcore_barrier()
        pltpu.sync_copy(accum.at[pl.ds(s * slc, slc)],
                        out_hbm.at[c].at[pl.ds(s * slc, slc)])

    return agg_kernel(g, src3d, dst3d, zblock)


def _tc_matmul1(xpad, W1):
    npad = xpad.shape[0]

    def body(x_ref, w_ref, h_ref):
        h_ref[...] = jnp.dot(
            x_ref[...], w_ref[...], preferred_element_type=jnp.float32
        )

    return pl.pallas_call(
        body,
        grid=(npad // BM,),
        in_specs=[
            pl.BlockSpec((BM, N_IN), lambda i: (i, 0)),
            pl.BlockSpec((N_IN, N_HID), lambda i: (0, 0)),
        ],
        out_specs=pl.BlockSpec((BM, N_HID), lambda i: (i, 0)),
        out_shape=jax.ShapeDtypeStruct((npad, N_HID), jnp.float32),
    )(xpad, W1)


def _tc_scale1(h1, degP):
    npad = h1.shape[0]

    def body(h_ref, dp_ref, g_ref, dis_ref):
        deg = jnp.sum(dp_ref[...], axis=0) + 1.0
        dis = lax.rsqrt(deg)[:, None]
        g_ref[...] = h_ref[...] * dis
        dis_ref[...] = dis

    return pl.pallas_call(
        body,
        grid=(npad // BM,),
        in_specs=[
            pl.BlockSpec((BM, N_HID), lambda i: (i, 0)),
            pl.BlockSpec((NW, BM), lambda i: (0, i)),
        ],
        out_specs=[
            pl.BlockSpec((BM, N_HID), lambda i: (i, 0)),
            pl.BlockSpec((BM, 1), lambda i: (i, 0)),
        ],
        out_shape=[
            jax.ShapeDtypeStruct((npad, N_HID), jnp.float32),
            jax.ShapeDtypeStruct((npad, 1), jnp.float32),
        ],
    )(h1, degP)


def _tc_layer2_in(S1, g1, dis, b1r, W2p):
    npad = g1.shape[0]

    def body(s_ref, g_ref, d_ref, b_ref, w_ref, o_ref):
        S = s_ref[0] + s_ref[1] + g_ref[...]
        h = jnp.maximum(d_ref[...] * S + b_ref[...], 0.0)
        o_ref[...] = (
            jnp.dot(h, w_ref[...], preferred_element_type=jnp.float32)
            * d_ref[...]
        )

    return pl.pallas_call(
        body,
        grid=(npad // BM,),
        in_specs=[
            pl.BlockSpec((NC, BM, N_HID), lambda i: (0, i, 0)),
            pl.BlockSpec((BM, N_HID), lambda i: (i, 0)),
            pl.BlockSpec((BM, 1), lambda i: (i, 0)),
            pl.BlockSpec((1, N_HID), lambda i: (0, 0)),
            pl.BlockSpec((N_HID, D2), lambda i: (0, 0)),
        ],
        out_specs=pl.BlockSpec((BM, D2), lambda i: (i, 0)),
        out_shape=jax.ShapeDtypeStruct((npad, D2), jnp.float32),
    )(S1, g1, dis, b1r, W2p)


def _tc_final(S2, g2, dis, b2p):
    npad = g2.shape[0]

    def body(s_ref, g_ref, d_ref, b_ref, o_ref):
        o_ref[...] = d_ref[...] * (s_ref[0] + s_ref[1] + g_ref[...]) + b_ref[...]

    return pl.pallas_call(
        body,
        grid=(npad // BM,),
        in_specs=[
            pl.BlockSpec((NC, BM, D2), lambda i: (0, i, 0)),
            pl.BlockSpec((BM, D2), lambda i: (i, 0)),
            pl.BlockSpec((BM, 1), lambda i: (i, 0)),
            pl.BlockSpec((1, D2), lambda i: (0, 0)),
        ],
        out_specs=pl.BlockSpec((BM, D2), lambda i: (i, 0)),
        out_shape=jax.ShapeDtypeStruct((npad, D2), jnp.float32),
    )(S2, g2, dis, b2p)


def kernel(x, edge_index, W1, b1, W2, b2):
    n = x.shape[0]
    src = edge_index[0].astype(jnp.int32)
    dst = edge_index[1].astype(jnp.int32)
    e = src.shape[0]

    npad = -(-(n + 1) // BM) * BM
    rows_pad = -(-(-(-e // K)) // (NW * QB)) * NW * QB
    ep = rows_pad * K
    pad = jnp.full((ep - e,), n, jnp.int32)
    src3d = jnp.concatenate([src, pad]).reshape(rows_pad // QB, QB * K)
    dst3d = jnp.concatenate([dst, pad]).reshape(rows_pad // QB, QB * K)
    dst2d = dst3d.reshape(rows_pad, K)

    xpad = jnp.pad(x, ((0, npad - n), (0, 0)))
    W2p = jnp.pad(W2, ((0, 0), (0, D2 - N_OUT)))
    b1r = b1.reshape(1, N_HID)
    b2p = jnp.pad(b2, (0, D2 - N_OUT)).reshape(1, D2)
    z1 = jnp.zeros((npad,), jnp.float32)
    z64 = jnp.zeros((npad, N_HID), jnp.float32)
    z16 = jnp.zeros((npad, D2), jnp.float32)

    degP = _sc_degree(dst2d, z1, npad)
    h1 = _tc_matmul1(xpad, W1)
    g1, dis = _tc_scale1(h1, degP)
    S1 = _sc_aggregate(g1, src3d, dst3d, z64)
    g2 = _tc_layer2_in(S1, g1, dis, b1r, W2p)
    S2 = _sc_aggregate(g2, src3d, dst3d, z16)
    out = _tc_final(S2, g2, dis, b2p)
    return out[:n, :N_OUT]


# speedup vs baseline: 1.9227x; 1.9227x over previous
import functools

import jax
import jax.numpy as jnp
from jax import lax
from jax.experimental import pallas as pl
from jax.experimental.pallas import tpu as pltpu
from jax.experimental.pallas import tpu_sc as plsc

N_IN = 128
N_HID = 64
N_OUT = 4
D2 = 16
K = 128
NC, NS = 2, 16
NW = NC * NS
BM = 512
QB = 1

_mesh = plsc.VectorSubcoreMesh(
    core_axis_name="c", subcore_axis_name="s", num_cores=NC, num_subcores=NS
)
_sc_params = pltpu.CompilerParams(
    needs_layout_passes=False, use_tc_tiling_on_sc=False
)


def _sc_degree(dst2d, zrow, npad):
    per_w = dst2d.shape[0] // NW

    @functools.partial(
        pl.kernel,
        out_type=jax.ShapeDtypeStruct((NW, npad), jnp.float32),
        mesh=_mesh,
        scratch_types=[
            pltpu.VMEM((per_w, K), jnp.int32),
            pltpu.VMEM((npad,), jnp.float32),
        ],
        compiler_params=_sc_params,
    )
    def deg_kernel(dst_hbm, z_hbm, out_hbm, idx_v, deg_v):
        c = lax.axis_index("c")
        s = lax.axis_index("s")
        w = c * NS + s
        pltpu.sync_copy(dst_hbm.at[pl.ds(w * per_w, per_w)], idx_v)
        pltpu.sync_copy(z_hbm, deg_v)
        ones = jnp.full((16,), 1.0, jnp.float32)

        @pl.loop(0, per_w)
        def _(j):
            for i in range(K // 16):
                plsc.addupdate_scatter(deg_v, [idx_v[j, pl.ds(i * 16, 16)]], ones)

        pltpu.sync_copy(deg_v, out_hbm.at[w])

    return deg_kernel(dst2d, zrow)


def _sc_aggregate(g, src3d, dst3d, zblock):
    nq = src3d.shape[0] // NW
    npad, d = g.shape
    slc = npad // NS

    @functools.partial(
        pl.kernel,
        out_type=jax.ShapeDtypeStruct((NC, npad, d), jnp.float32),
        mesh=_mesh,
        scratch_types=[
            pltpu.VMEM((nq, QB * K), jnp.int32),
            pltpu.VMEM((nq, QB * K), jnp.int32),
            pltpu.VMEM((QB * K, d), jnp.float32),
            pltpu.VMEM_SHARED((npad, d), jnp.float32),
            pltpu.VMEM_SHARED((npad, d), jnp.float32),
            pltpu.SemaphoreType.DMA,
        ],
        compiler_params=_sc_params,
    )
    def agg_kernel(g_hbm, src_hbm, dst_hbm, z_hbm, out_hbm,
                   sidx_v, didx_v, rows_v, accum, g_sh, gsem):
        c = lax.axis_index("c")
        s = lax.axis_index("s")
        w = c * NS + s
        pltpu.sync_copy(src_hbm.at[pl.ds(w * nq, nq)], sidx_v)
        pltpu.sync_copy(dst_hbm.at[pl.ds(w * nq, nq)], didx_v)
        pltpu.sync_copy(g_hbm.at[pl.ds(s * slc, slc)],
                        g_sh.at[pl.ds(s * slc, slc)])
        pltpu.sync_copy(z_hbm.at[pl.ds(s * slc, slc)],
                        accum.at[pl.ds(s * slc, slc)])
        plsc.subcore_barrier()

        @pl.loop(0, nq)
        def _(j):
            pltpu.async_copy(g_sh.at[sidx_v.at[j]], rows_v, gsem).wait()
            pltpu.sync_copy(rows_v, accum.at[didx_v.at[j]], add=True)

        plsc.subcore_barrier()
        pltpu.sync_copy(accum.at[pl.ds(s * slc, slc)],
                        out_hbm.at[c].at[pl.ds(s * slc, slc)])

    return agg_kernel(g, src3d, dst3d, zblock)


def _tc_matmul1(xpad, W1):
    npad = xpad.shape[0]

    def body(x_ref, w_ref, h_ref):
        h_ref[...] = jnp.dot(
            x_ref[...], w_ref[...], preferred_element_type=jnp.float32
        )

    return pl.pallas_call(
        body,
        grid=(npad // BM,),
        in_specs=[
            pl.BlockSpec((BM, N_IN), lambda i: (i, 0)),
            pl.BlockSpec((N_IN, N_HID), lambda i: (0, 0)),
        ],
        out_specs=pl.BlockSpec((BM, N_HID), lambda i: (i, 0)),
        out_shape=jax.ShapeDtypeStruct((npad, N_HID), jnp.float32),
    )(xpad, W1)


def _tc_scale1(h1, degP):
    npad = h1.shape[0]

    def body(h_ref, dp_ref, g_ref, dis_ref):
        deg = jnp.sum(dp_ref[...], axis=0) + 1.0
        dis = lax.rsqrt(deg)[:, None]
        g_ref[...] = h_ref[...] * dis
        dis_ref[...] = dis

    return pl.pallas_call(
        body,
        grid=(npad // BM,),
        in_specs=[
            pl.BlockSpec((BM, N_HID), lambda i: (i, 0)),
            pl.BlockSpec((NW, BM), lambda i: (0, i)),
        ],
        out_specs=[
            pl.BlockSpec((BM, N_HID), lambda i: (i, 0)),
            pl.BlockSpec((BM, 1), lambda i: (i, 0)),
        ],
        out_shape=[
            jax.ShapeDtypeStruct((npad, N_HID), jnp.float32),
            jax.ShapeDtypeStruct((npad, 1), jnp.float32),
        ],
    )(h1, degP)


def _tc_layer2_in(S1, g1, dis, b1r, W2p):
    npad = g1.shape[0]

    def body(s_ref, g_ref, d_ref, b_ref, w_ref, o_ref):
        S = s_ref[0] + s_ref[1] + g_ref[...]
        h = jnp.maximum(d_ref[...] * S + b_ref[...], 0.0)
        o_ref[...] = (
            jnp.dot(h, w_ref[...], preferred_element_type=jnp.float32)
            * d_ref[...]
        )

    return pl.pallas_call(
        body,
        grid=(npad // BM,),
        in_specs=[
            pl.BlockSpec((NC, BM, N_HID), lambda i: (0, i, 0)),
            pl.BlockSpec((BM, N_HID), lambda i: (i, 0)),
            pl.BlockSpec((BM, 1), lambda i: (i, 0)),
            pl.BlockSpec((1, N_HID), lambda i: (0, 0)),
            pl.BlockSpec((N_HID, D2), lambda i: (0, 0)),
        ],
        out_specs=pl.BlockSpec((BM, D2), lambda i: (i, 0)),
        out_shape=jax.ShapeDtypeStruct((npad, D2), jnp.float32),
    )(S1, g1, dis, b1r, W2p)


def _tc_final(S2, g2, dis, b2p):
    npad = g2.shape[0]

    def body(s_ref, g_ref, d_ref, b_ref, o_ref):
        o_ref[...] = d_ref[...] * (s_ref[0] + s_ref[1] + g_ref[...]) + b_ref[...]

    return pl.pallas_call(
        body,
        grid=(npad // BM,),
        in_specs=[
            pl.BlockSpec((NC, BM, D2), lambda i: (0, i, 0)),
            pl.BlockSpec((BM, D2), lambda i: (i, 0)),
            pl.BlockSpec((BM, 1), lambda i: (i, 0)),
            pl.BlockSpec((1, D2), lambda i: (0, 0)),
        ],
        out_specs=pl.BlockSpec((BM, D2), lambda i: (i, 0)),
        out_shape=jax.ShapeDtypeStruct((npad, D2), jnp.float32),
    )(S2, g2, dis, b2p)


def kernel(x, edge_index, W1, b1, W2, b2):
    n = x.shape[0]
    src = edge_index[0].astype(jnp.int32)
    dst = edge_index[1].astype(jnp.int32)
    e = src.shape[0]

    npad = -(-(n + 1) // BM) * BM
    rows_pad = -(-(-(-e // K)) // (NW * QB)) * NW * QB
    ep = rows_pad * K
    pad = jnp.full((ep - e,), n, jnp.int32)
    src3d = jnp.concatenate([src, pad]).reshape(rows_pad // QB, QB * K)
    dst3d = jnp.concatenate([dst, pad]).reshape(rows_pad // QB, QB * K)
    dst2d = dst3d.reshape(rows_pad, K)

    xpad = jnp.pad(x, ((0, npad - n), (0, 0)))
    W2p = jnp.pad(W2, ((0, 0), (0, D2 - N_OUT)))
    b1r = b1.reshape(1, N_HID)
    b2p = jnp.pad(b2, (0, D2 - N_OUT)).reshape(1, D2)
    z1 = jnp.zeros((npad,), jnp.float32)
    z64 = jnp.zeros((npad, N_HID), jnp.float32)
    z16 = jnp.zeros((npad, D2), jnp.float32)

    degP = _sc_degree(dst2d, z1, npad)
    h1 = _tc_matmul1(xpad, W1)
    g1, dis = _tc_scale1(h1, degP)
    S1 = _sc_aggregate(g1, src3d, dst3d, z64)
    g2 = _tc_layer2_in(S1, g1, dis, b1r, W2p)
    S2 = _sc_aggregate(g2, src3d, dst3d, z16)
    out = _tc_final(S2, g2, dis, b2p)
    return out[:n, :N_OUT]


# speedup vs baseline: 1.9314x; 1.0045x over previous
import functools

import jax
import jax.numpy as jnp
from jax import lax
from jax.experimental import pallas as pl
from jax.experimental.pallas import tpu as pltpu
from jax.experimental.pallas import tpu_sc as plsc

N_IN = 128
N_HID = 64
N_OUT = 4
D2 = 16
K = 128
NC, NS = 2, 16
NW = NC * NS
BM = 512
QB = 2

_mesh = plsc.VectorSubcoreMesh(
    core_axis_name="c", subcore_axis_name="s", num_cores=NC, num_subcores=NS
)
_sc_params = pltpu.CompilerParams(
    needs_layout_passes=False, use_tc_tiling_on_sc=False
)


def _sc_degree(dst2d, zrow, npad):
    per_w = dst2d.shape[0] // NW

    @functools.partial(
        pl.kernel,
        out_type=jax.ShapeDtypeStruct((NW, npad), jnp.float32),
        mesh=_mesh,
        scratch_types=[
            pltpu.VMEM((per_w, K), jnp.int32),
            pltpu.VMEM((npad,), jnp.float32),
        ],
        compiler_params=_sc_params,
    )
    def deg_kernel(dst_hbm, z_hbm, out_hbm, idx_v, deg_v):
        c = lax.axis_index("c")
        s = lax.axis_index("s")
        w = c * NS + s
        pltpu.sync_copy(dst_hbm.at[pl.ds(w * per_w, per_w)], idx_v)
        pltpu.sync_copy(z_hbm, deg_v)
        ones = jnp.full((16,), 1.0, jnp.float32)

        @pl.loop(0, per_w)
        def _(j):
            for i in range(K // 16):
                plsc.addupdate_scatter(deg_v, [idx_v[j, pl.ds(i * 16, 16)]], ones)

        pltpu.sync_copy(deg_v, out_hbm.at[w])

    return deg_kernel(dst2d, zrow)


def _sc_aggregate(g, src3d, dst3d, zblock):
    nq = src3d.shape[0] // NW
    npad, d = g.shape
    slc = npad // NS

    @functools.partial(
        pl.kernel,
        out_type=jax.ShapeDtypeStruct((NC, npad, d), jnp.float32),
        mesh=_mesh,
        scratch_types=[
            pltpu.VMEM((nq, QB * K), jnp.int32),
            pltpu.VMEM((nq, QB * K), jnp.int32),
            pltpu.VMEM((QB * K, d), jnp.float32),
            pltpu.VMEM_SHARED((npad, d), jnp.float32),
            pltpu.VMEM_SHARED((npad, d), jnp.float32),
            pltpu.SemaphoreType.DMA,
        ],
        compiler_params=_sc_params,
    )
    def agg_kernel(g_hbm, src_hbm, dst_hbm, z_hbm, out_hbm,
                   sidx_v, didx_v, rows_v, accum, g_sh, gsem):
        c = lax.axis_index("c")
        s = lax.axis_index("s")
        w = c * NS + s
        pltpu.sync_copy(src_hbm.at[pl.ds(w * nq, nq)], sidx_v)
        pltpu.sync_copy(dst_hbm.at[pl.ds(w * nq, nq)], didx_v)
        pltpu.sync_copy(g_hbm.at[pl.ds(s * slc, slc)],
                        g_sh.at[pl.ds(s * slc, slc)])
        pltpu.sync_copy(z_hbm.at[pl.ds(s * slc, slc)],
                        accum.at[pl.ds(s * slc, slc)])
        plsc.subcore_barrier()

        @pl.loop(0, nq)
        def _(j):
            pltpu.async_copy(g_sh.at[sidx_v.at[j]], rows_v, gsem).wait()
            pltpu.sync_copy(rows_v, accum.at[didx_v.at[j]], add=True)

        plsc.subcore_barrier()
        pltpu.sync_copy(accum.at[pl.ds(s * slc, slc)],
                        out_hbm.at[c].at[pl.ds(s * slc, slc)])

    return agg_kernel(g, src3d, dst3d, zblock)


def _tc_matmul1(xpad, W1):
    npad = xpad.shape[0]

    def body(x_ref, w_ref, h_ref):
        h_ref[...] = jnp.dot(
            x_ref[...], w_ref[...], preferred_element_type=jnp.float32
        )

    return pl.pallas_call(
        body,
        grid=(npad // BM,),
        in_specs=[
            pl.BlockSpec((BM, N_IN), lambda i: (i, 0)),
            pl.BlockSpec((N_IN, N_HID), lambda i: (0, 0)),
        ],
        out_specs=pl.BlockSpec((BM, N_HID), lambda i: (i, 0)),
        out_shape=jax.ShapeDtypeStruct((npad, N_HID), jnp.float32),
    )(xpad, W1)


def _tc_scale1(h1, degP):
    npad = h1.shape[0]

    def body(h_ref, dp_ref, g_ref, dis_ref):
        deg = jnp.sum(dp_ref[...], axis=0) + 1.0
        dis = lax.rsqrt(deg)[:, None]
        g_ref[...] = h_ref[...] * dis
        dis_ref[...] = dis

    return pl.pallas_call(
        body,
        grid=(npad // BM,),
        in_specs=[
            pl.BlockSpec((BM, N_HID), lambda i: (i, 0)),
            pl.BlockSpec((NW, BM), lambda i: (0, i)),
        ],
        out_specs=[
            pl.BlockSpec((BM, N_HID), lambda i: (i, 0)),
            pl.BlockSpec((BM, 1), lambda i: (i, 0)),
        ],
        out_shape=[
            jax.ShapeDtypeStruct((npad, N_HID), jnp.float32),
            jax.ShapeDtypeStruct((npad, 1), jnp.float32),
        ],
    )(h1, degP)


def _tc_layer2_in(S1, g1, dis, b1r, W2p):
    npad = g1.shape[0]

    def body(s_ref, g_ref, d_ref, b_ref, w_ref, o_ref):
        S = s_ref[0] + s_ref[1] + g_ref[...]
        h = jnp.maximum(d_ref[...] * S + b_ref[...], 0.0)
        o_ref[...] = (
            jnp.dot(h, w_ref[...], preferred_element_type=jnp.float32)
            * d_ref[...]
        )

    return pl.pallas_call(
        body,
        grid=(npad // BM,),
        in_specs=[
            pl.BlockSpec((NC, BM, N_HID), lambda i: (0, i, 0)),
            pl.BlockSpec((BM, N_HID), lambda i: (i, 0)),
            pl.BlockSpec((BM, 1), lambda i: (i, 0)),
            pl.BlockSpec((1, N_HID), lambda i: (0, 0)),
            pl.BlockSpec((N_HID, D2), lambda i: (0, 0)),
        ],
        out_specs=pl.BlockSpec((BM, D2), lambda i: (i, 0)),
        out_shape=jax.ShapeDtypeStruct((npad, D2), jnp.float32),
    )(S1, g1, dis, b1r, W2p)


def _tc_final(S2, g2, dis, b2p):
    npad = g2.shape[0]

    def body(s_ref, g_ref, d_ref, b_ref, o_ref):
        o_ref[...] = d_ref[...] * (s_ref[0] + s_ref[1] + g_ref[...]) + b_ref[...]

    return pl.pallas_call(
        body,
        grid=(npad // BM,),
        in_specs=[
            pl.BlockSpec((NC, BM, D2), lambda i: (0, i, 0)),
            pl.BlockSpec((BM, D2), lambda i: (i, 0)),
            pl.BlockSpec((BM, 1), lambda i: (i, 0)),
            pl.BlockSpec((1, D2), lambda i: (0, 0)),
        ],
        out_specs=pl.BlockSpec((BM, D2), lambda i: (i, 0)),
        out_shape=jax.ShapeDtypeStruct((npad, D2), jnp.float32),
    )(S2, g2, dis, b2p)


def kernel(x, edge_index, W1, b1, W2, b2):
    n = x.shape[0]
    src = edge_index[0].astype(jnp.int32)
    dst = edge_index[1].astype(jnp.int32)
    e = src.shape[0]

    npad = -(-(n + 1) // BM) * BM
    rows_pad = -(-(-(-e // K)) // (NW * QB)) * NW * QB
    ep = rows_pad * K
    pad = jnp.full((ep - e,), n, jnp.int32)
    src3d = jnp.concatenate([src, pad]).reshape(rows_pad // QB, QB * K)
    dst3d = jnp.concatenate([dst, pad]).reshape(rows_pad // QB, QB * K)
    dst2d = dst3d.reshape(rows_pad, K)

    xpad = jnp.pad(x, ((0, npad - n), (0, 0)))
    W2p = jnp.pad(W2, ((0, 0), (0, D2 - N_OUT)))
    b1r = b1.reshape(1, N_HID)
    b2p = jnp.pad(b2, (0, D2 - N_OUT)).reshape(1, D2)
    z1 = jnp.zeros((npad,), jnp.float32)
    z64 = jnp.zeros((npad, N_HID), jnp.float32)
    z16 = jnp.zeros((npad, D2), jnp.float32)

    degP = _sc_degree(dst2d, z1, npad)
    h1 = _tc_matmul1(xpad, W1)
    g1, dis = _tc_scale1(h1, degP)
    S1 = _sc_aggregate(g1, src3d, dst3d, z64)
    g2 = _tc_layer2_in(S1, g1, dis, b1r, W2p)
    S2 = _sc_aggregate(g2, src3d, dst3d, z16)
    out = _tc_final(S2, g2, dis, b2p)
    return out[:n, :N_OUT]


# speedup vs baseline: 1.9556x; 1.0125x over previous
import functools

import jax
import jax.numpy as jnp
from jax import lax
from jax.experimental import pallas as pl
from jax.experimental.pallas import tpu as pltpu
from jax.experimental.pallas import tpu_sc as plsc

N_IN = 128
N_HID = 64
N_OUT = 4
D2 = 16
K = 128
NC, NS = 2, 16
NW = NC * NS
BM = 512
QB = 2

_mesh = plsc.VectorSubcoreMesh(
    core_axis_name="c", subcore_axis_name="s", num_cores=NC, num_subcores=NS
)
_sc_params = pltpu.CompilerParams(
    needs_layout_passes=False, use_tc_tiling_on_sc=False
)


def _sc_degree(dst2d, zrow, npad):
    per_w = dst2d.shape[0] // NW

    @functools.partial(
        pl.kernel,
        out_type=jax.ShapeDtypeStruct((NW, npad), jnp.float32),
        mesh=_mesh,
        scratch_types=[
            pltpu.VMEM((per_w, K), jnp.int32),
            pltpu.VMEM((npad,), jnp.float32),
        ],
        compiler_params=_sc_params,
    )
    def deg_kernel(dst_hbm, z_hbm, out_hbm, idx_v, deg_v):
        c = lax.axis_index("c")
        s = lax.axis_index("s")
        w = c * NS + s
        pltpu.sync_copy(dst_hbm.at[pl.ds(w * per_w, per_w)], idx_v)
        pltpu.sync_copy(z_hbm, deg_v)
        ones = jnp.full((16,), 1.0, jnp.float32)

        @pl.loop(0, per_w)
        def _(j):
            for i in range(K // 16):
                plsc.addupdate_scatter(deg_v, [idx_v[j, pl.ds(i * 16, 16)]], ones)

        pltpu.sync_copy(deg_v, out_hbm.at[w])

    return deg_kernel(dst2d, zrow)


def _sc_aggregate(g, src3d, dst3d, zblock):
    nq = src3d.shape[0] // NW
    npad, d = g.shape
    slc = npad // NS

    @functools.partial(
        pl.kernel,
        out_type=jax.ShapeDtypeStruct((NC, npad, d), jnp.float32),
        mesh=_mesh,
        scratch_types=[
            pltpu.VMEM((nq, QB * K), jnp.int32),
            pltpu.VMEM((nq, QB * K), jnp.int32),
            pltpu.VMEM((QB * K, d), jnp.float32),
            pltpu.VMEM_SHARED((npad, d), jnp.float32),
            pltpu.VMEM_SHARED((npad, d), jnp.float32),
            pltpu.SemaphoreType.DMA,
        ],
        compiler_params=_sc_params,
    )
    def agg_kernel(g_hbm, src_hbm, dst_hbm, z_hbm, out_hbm,
                   sidx_v, didx_v, rows_v, accum, g_sh, gsem):
        c = lax.axis_index("c")
        s = lax.axis_index("s")
        w = c * NS + s
        pltpu.sync_copy(src_hbm.at[pl.ds(w * nq, nq)], sidx_v)
        pltpu.sync_copy(dst_hbm.at[pl.ds(w * nq, nq)], didx_v)
        pltpu.sync_copy(g_hbm.at[pl.ds(s * slc, slc)],
                        g_sh.at[pl.ds(s * slc, slc)])
        pltpu.sync_copy(z_hbm.at[pl.ds(s * slc, slc)],
                        accum.at[pl.ds(s * slc, slc)])
        plsc.subcore_barrier()

        @pl.loop(0, nq)
        def _(j):
            pltpu.async_copy(g_sh.at[sidx_v.at[j]], rows_v, gsem).wait()
            pltpu.sync_copy(rows_v, accum.at[didx_v.at[j]], add=True)

        plsc.subcore_barrier()
        pltpu.sync_copy(accum.at[pl.ds(s * slc, slc)],
                        out_hbm.at[c].at[pl.ds(s * slc, slc)])

    return agg_kernel(g, src3d, dst3d, zblock)


def _tc_layer1(xpad, W1, degP):
    npad = xpad.shape[0]

    def body(x_ref, w_ref, dp_ref, g_ref, dis_ref):
        deg = jnp.sum(dp_ref[...], axis=0) + 1.0
        dis = lax.rsqrt(deg)[:, None]
        h = jnp.dot(x_ref[...], w_ref[...], preferred_element_type=jnp.float32)
        g_ref[...] = h * dis
        dis_ref[...] = dis

    return pl.pallas_call(
        body,
        grid=(npad // BM,),
        in_specs=[
            pl.BlockSpec((BM, N_IN), lambda i: (i, 0)),
            pl.BlockSpec((N_IN, N_HID), lambda i: (0, 0)),
            pl.BlockSpec((NW, BM), lambda i: (0, i)),
        ],
        out_specs=[
            pl.BlockSpec((BM, N_HID), lambda i: (i, 0)),
            pl.BlockSpec((BM, 1), lambda i: (i, 0)),
        ],
        out_shape=[
            jax.ShapeDtypeStruct((npad, N_HID), jnp.float32),
            jax.ShapeDtypeStruct((npad, 1), jnp.float32),
        ],
    )(xpad, W1, degP)


def _tc_layer2_in(S1, g1, dis, b1r, W2p):
    npad = g1.shape[0]

    def body(s_ref, g_ref, d_ref, b_ref, w_ref, o_ref):
        S = s_ref[0] + s_ref[1] + g_ref[...]
        h = jnp.maximum(d_ref[...] * S + b_ref[...], 0.0)
        o_ref[...] = (
            jnp.dot(h, w_ref[...], preferred_element_type=jnp.float32)
            * d_ref[...]
        )

    return pl.pallas_call(
        body,
        grid=(npad // BM,),
        in_specs=[
            pl.BlockSpec((NC, BM, N_HID), lambda i: (0, i, 0)),
            pl.BlockSpec((BM, N_HID), lambda i: (i, 0)),
            pl.BlockSpec((BM, 1), lambda i: (i, 0)),
            pl.BlockSpec((1, N_HID), lambda i: (0, 0)),
            pl.BlockSpec((N_HID, D2), lambda i: (0, 0)),
        ],
        out_specs=pl.BlockSpec((BM, D2), lambda i: (i, 0)),
        out_shape=jax.ShapeDtypeStruct((npad, D2), jnp.float32),
    )(S1, g1, dis, b1r, W2p)


def _tc_final(S2, g2, dis, b2p, n):
    bm = 400
    assert n % bm == 0

    def body(s_ref, g_ref, d_ref, b_ref, o_ref):
        full = d_ref[...] * (s_ref[0] + s_ref[1] + g_ref[...]) + b_ref[...]
        o_ref[...] = full[:, :N_OUT]

    return pl.pallas_call(
        body,
        grid=(n // bm,),
        in_specs=[
            pl.BlockSpec((NC, bm, D2), lambda i: (0, i, 0)),
            pl.BlockSpec((bm, D2), lambda i: (i, 0)),
            pl.BlockSpec((bm, 1), lambda i: (i, 0)),
            pl.BlockSpec((1, D2), lambda i: (0, 0)),
        ],
        out_specs=pl.BlockSpec((bm, N_OUT), lambda i: (i, 0)),
        out_shape=jax.ShapeDtypeStruct((n, N_OUT), jnp.float32),
    )(S2, g2, dis, b2p)


def kernel(x, edge_index, W1, b1, W2, b2):
    n = x.shape[0]
    src = edge_index[0].astype(jnp.int32)
    dst = edge_index[1].astype(jnp.int32)
    e = src.shape[0]

    npad = -(-(n + 1) // BM) * BM
    rows_pad = -(-(-(-e // K)) // (NW * QB)) * NW * QB
    ep = rows_pad * K
    pad = jnp.full((ep - e,), n, jnp.int32)
    src3d = jnp.concatenate([src, pad]).reshape(rows_pad // QB, QB * K)
    dst3d = jnp.concatenate([dst, pad]).reshape(rows_pad // QB, QB * K)
    dst2d = dst3d.reshape(rows_pad, K)

    xpad = jnp.pad(x, ((0, npad - n), (0, 0)))
    W2p = jnp.pad(W2, ((0, 0), (0, D2 - N_OUT)))
    b1r = b1.reshape(1, N_HID)
    b2p = jnp.pad(b2, (0, D2 - N_OUT)).reshape(1, D2)
    z1 = jnp.zeros((npad,), jnp.float32)
    z64 = jnp.zeros((npad, N_HID), jnp.float32)
    z16 = jnp.zeros((npad, D2), jnp.float32)

    degP = _sc_degree(dst2d, z1, npad)
    g1, dis = _tc_layer1(xpad, W1, degP)
    S1 = _sc_aggregate(g1, src3d, dst3d, z64)
    g2 = _tc_layer2_in(S1, g1, dis, b1r, W2p)
    S2 = _sc_aggregate(g2, src3d, dst3d, z16)
    return _tc_final(S2, g2, dis, b2p, n)
